# Initial kernel scaffold; baseline (speedup 1.0000x reference)
import jax
import jax.numpy as jnp
from jax.experimental import pallas as pl


def kernel(X, TE, adj_mx, W_in1, b_in1, W_in2, b_in2, enc_W_ru, enc_b_ru, enc_W_c, enc_b_c, dec_W_ru, dec_b_ru, dec_W_c, dec_b_c, W_out1, b_out1, W_out2, b_out2):
    raise NotImplementedError("write your pallas kernel here")


# speedup vs baseline: 17.5986x; 17.5986x over previous
import jax
import jax.numpy as jnp
from jax.experimental import pallas as pl
from jax.experimental.pallas import tpu as pltpu

P = 12
Q = 12
D = 64
N = 207
B = 16
NP = 256
M = 5
F32 = jnp.float32


def _dcgru_kernel(Xp_r, tod_r, adj_r, adjT_r,
                  W1x_r, W1t_r, b1_r, W2T_r, b2_r,
                  WruTe_r, brue_r, WcTe_r, bce_r,
                  WruTd_r, brud_r, WcTd_r, bcd_r,
                  Wo1T_r, bo1_r, w2t_r, bo2_r,
                  out_r, h_r):
    dot = lambda a, b: jnp.dot(a, b, preferred_element_type=F32)

    A = adj_r[...]
    d1 = jnp.sum(A, axis=1, keepdims=True)
    R1 = jnp.where(d1 > 0, 1.0 / d1, 0.0) * A
    AT = adjT_r[...]
    d2 = jnp.sum(AT, axis=1, keepdims=True)
    R2 = jnp.where(d2 > 0, 1.0 / d2, 0.0) * AT

    h_r[...] = jnp.zeros((B * D, NP), F32)

    def dconv(xh, tb, WT, bcol):
        m1 = dot(xh, R1)
        m2 = 2.0 * dot(m1, R1) - xh
        m3 = dot(xh, R2)
        m4 = 2.0 * dot(m3, R2) - xh
        mats = (xh, m1, m2, m3, m4)
        outs = []
        for b in range(B):
            Zb = jnp.concatenate(
                [m[b * tb:(b + 1) * tb, :] for m in mats], axis=0)
            outs.append(dot(WT, Zb) + bcol)
        return outs

    def gates(ru_list, h, cWT, cb, x_pieces, tb):
        r = jax.nn.sigmoid(jnp.concatenate([rb[:D] for rb in ru_list], 0))
        u = jax.nn.sigmoid(jnp.concatenate([rb[D:] for rb in ru_list], 0))
        rh = r * h
        if x_pieces is None:
            xh2 = rh
        else:
            pieces = []
            for b in range(B):
                pieces.append(x_pieces[b])
                pieces.append(rh[b * D:(b + 1) * D, :])
            xh2 = jnp.concatenate(pieces, 0)
        c_list = dconv(xh2, tb, cWT, cb)
        c = jnp.tanh(jnp.concatenate(c_list, 0))
        return u * h + (1.0 - u) * c

    W1x = W1x_r[...]
    W1t = W1t_r[...]
    b1 = b1_r[...]
    W2T = W2T_r[...]
    b2 = b2_r[...]
    WruTe = WruTe_r[...]
    brue = brue_r[...]
    WcTe = WcTe_r[...]
    bce = bce_r[...]

    def enc_body(p, carry):
        h = h_r[...]
        xr = Xp_r[p]
        tr = tod_r[p]
        xs = []
        for b in range(B):
            arg = W1x * xr[b:b + 1, :] + W1t * tr[b:b + 1, :] + b1
            xs.append(dot(W2T, jnp.maximum(arg, 0.0)) + b2)
        pieces = []
        for b in range(B):
            pieces.append(xs[b])
            pieces.append(h[b * D:(b + 1) * D, :])
        xh = jnp.concatenate(pieces, 0)
        ru_list = dconv(xh, 2 * D, WruTe, brue)
        h_r[...] = gates(ru_list, h, WcTe, bce, xs, 2 * D)
        return carry

    jax.lax.fori_loop(0, P, enc_body, 0)

    WruTd = WruTd_r[...]
    brud = brud_r[...]
    WcTd = WcTd_r[...]
    bcd = bcd_r[...]
    Wo1T = Wo1T_r[...]
    bo1 = bo1_r[...]
    w2t = w2t_r[...]
    bo2 = bo2_r[...]

    def dec_body(q, carry):
        h = h_r[...]
        ru_list = dconv(h, D, WruTd, brud)
        h2 = gates(ru_list, h, WcTd, bcd, None, D)
        h_r[...] = h2
        o_pieces = [
            jnp.maximum(dot(Wo1T, h2[b * D:(b + 1) * D, :]) + bo1, 0.0)
            for b in range(B)
        ]
        o1 = jnp.concatenate(o_pieces, 0) * w2t
        srows = [
            jnp.sum(o1[b * D:(b + 1) * D, :], axis=0, keepdims=True)
            for b in range(B)
        ]
        out_r[q] = jnp.concatenate(srows, 0) + bo2
        return carry

    jax.lax.fori_loop(0, Q, dec_body, 0)


def _dcgru_call(Xp, todp, adjp, adjTp, W1x, W1t, b1, W2T, b2,
                WruTe, brue, WcTe, bce, WruTd, brud, WcTd, bcd,
                Wo1T, bo1, w2t, bo2, interpret=False):
    return pl.pallas_call(
        _dcgru_kernel,
        out_shape=jax.ShapeDtypeStruct((Q, B, NP), F32),
        scratch_shapes=[pltpu.VMEM((B * D, NP), F32)],
        interpret=interpret,
    )(Xp, todp, adjp, adjTp, W1x, W1t, b1, W2T, b2,
      WruTe, brue, WcTe, bce, WruTd, brud, WcTd, bcd,
      Wo1T, bo1, w2t, bo2)


def kernel(X, TE, adj_mx, W_in1, b_in1, W_in2, b_in2,
           enc_W_ru, enc_b_ru, enc_W_c, enc_b_c,
           dec_W_ru, dec_b_ru, dec_W_c, dec_b_c,
           W_out1, b_out1, W_out2, b_out2):
    f32 = F32
    ts = 2 * D

    Xsq = X[..., 0].astype(f32)
    Xp = jnp.zeros((P, B, NP), f32).at[:, :, :N].set(Xsq.transpose(1, 0, 2))
    tod = TE[:, :P, -1].astype(f32) / (12.0 * 24.0)
    todp = jnp.broadcast_to(tod.T[:, :, None], (P, B, NP))

    adjp = jnp.zeros((NP, NP), f32).at[:N, :N].set(adj_mx)
    adjTp = jnp.zeros((NP, NP), f32).at[:N, :N].set(adj_mx.T)

    W1x = W_in1[0][:, None]
    W1t = W_in1[1][:, None]
    b1 = b_in1[:, None]
    W2T = W_in2.T
    b2 = b_in2[:, None]

    WruTe = enc_W_ru.reshape(ts, M, ts).transpose(2, 1, 0).reshape(ts, M * ts)
    brue = enc_b_ru[:, None]
    WcTe = enc_W_c.reshape(ts, M, D).transpose(2, 1, 0).reshape(D, M * ts)
    bce = enc_b_c[:, None]
    WruTd = dec_W_ru.reshape(ts, M, ts)[D:].transpose(2, 1, 0).reshape(ts, M * D)
    brud = dec_b_ru[:, None]
    WcTd = dec_W_c.reshape(ts, M, D)[D:].transpose(2, 1, 0).reshape(D, M * D)
    bcd = dec_b_c[:, None]

    Wo1T = W_out1.T
    bo1 = b_out1[:, None]
    w2t = jnp.tile(W_out2[:, 0], B)[:, None]
    bo2 = b_out2.reshape(1, 1)

    out = _dcgru_call(Xp, todp, adjp, adjTp, W1x, W1t, b1, W2T, b2,
                      WruTe, brue, WcTe, bce, WruTd, brud, WcTd, bcd,
                      Wo1T, bo1, w2t, bo2)
    return out.transpose(1, 0, 2)[:, :, :N, None]


# speedup vs baseline: 18.3682x; 1.0437x over previous
import jax
import jax.numpy as jnp
from jax.experimental import pallas as pl
from jax.experimental.pallas import tpu as pltpu

P = 12
Q = 12
D = 64
N = 207
B = 16
NP = 256
M = 5
F32 = jnp.float32
BF16 = jnp.bfloat16


def _dcgru_kernel(Xp_r, tod_r, adj_r, adjT_r,
                  W1x_r, W1t_r, b1_r, W2T_r, b2_r,
                  WruTe_r, brue_r, WcTe_r, bce_r,
                  WruTd_r, brud_r, WcTd_r, bcd_r,
                  Wo1T_r, bo1_r, w2t_r, bo2_r,
                  out_r, h_r):
    dot = lambda a, b: jnp.dot(a, b, preferred_element_type=F32)

    A = adj_r[...]
    d1 = jnp.sum(A, axis=1, keepdims=True)
    R1 = (jnp.where(d1 > 0, 1.0 / d1, 0.0) * A).astype(BF16)
    AT = adjT_r[...]
    d2 = jnp.sum(AT, axis=1, keepdims=True)
    R2 = (jnp.where(d2 > 0, 1.0 / d2, 0.0) * AT).astype(BF16)

    h_r[...] = jnp.zeros((B * D, NP), F32)

    def dconv(xh, tb, WT, bcol):
        m0 = xh.astype(BF16)
        m1 = dot(m0, R1)
        m1b = m1.astype(BF16)
        m2b = (2.0 * dot(m1b, R1) - xh).astype(BF16)
        m3 = dot(m0, R2)
        m3b = m3.astype(BF16)
        m4b = (2.0 * dot(m3b, R2) - xh).astype(BF16)
        mats = (m0, m1b, m2b, m3b, m4b)
        outs = []
        for b in range(B):
            Zb = jnp.concatenate(
                [m[b * tb:(b + 1) * tb, :] for m in mats], axis=0)
            outs.append(dot(WT, Zb) + bcol)
        return outs

    def gates(ru_list, h, cWT, cb, x_pieces, tb):
        r = jax.nn.sigmoid(jnp.concatenate([rb[:D] for rb in ru_list], 0))
        u = jax.nn.sigmoid(jnp.concatenate([rb[D:] for rb in ru_list], 0))
        rh = r * h
        if x_pieces is None:
            xh2 = rh
        else:
            pieces = []
            for b in range(B):
                pieces.append(x_pieces[b])
                pieces.append(rh[b * D:(b + 1) * D, :])
            xh2 = jnp.concatenate(pieces, 0)
        c_list = dconv(xh2, tb, cWT, cb)
        c = jnp.tanh(jnp.concatenate(c_list, 0))
        return u * h + (1.0 - u) * c

    W1x = W1x_r[...]
    W1t = W1t_r[...]
    b1 = b1_r[...]
    W2T = W2T_r[...]
    b2 = b2_r[...]
    WruTe = WruTe_r[...]
    brue = brue_r[...]
    WcTe = WcTe_r[...]
    bce = bce_r[...]

    def enc_body(p, carry):
        h = h_r[...]
        xr = Xp_r[p]
        tr = tod_r[p]
        xs = []
        for b in range(B):
            arg = W1x * xr[b:b + 1, :] + W1t * tr[b:b + 1, :] + b1
            xs.append(dot(W2T, jnp.maximum(arg, 0.0)) + b2)
        pieces = []
        for b in range(B):
            pieces.append(xs[b])
            pieces.append(h[b * D:(b + 1) * D, :])
        xh = jnp.concatenate(pieces, 0)
        ru_list = dconv(xh, 2 * D, WruTe, brue)
        h_r[...] = gates(ru_list, h, WcTe, bce, xs, 2 * D)
        return carry

    jax.lax.fori_loop(0, P, enc_body, 0)

    WruTd = WruTd_r[...]
    brud = brud_r[...]
    WcTd = WcTd_r[...]
    bcd = bcd_r[...]
    Wo1T = Wo1T_r[...]
    bo1 = bo1_r[...]
    w2t = w2t_r[...]
    bo2 = bo2_r[...]

    def dec_body(q, carry):
        h = h_r[...]
        ru_list = dconv(h, D, WruTd, brud)
        h2 = gates(ru_list, h, WcTd, bcd, None, D)
        h_r[...] = h2
        o_pieces = [
            jnp.maximum(dot(Wo1T, h2[b * D:(b + 1) * D, :]) + bo1, 0.0)
            for b in range(B)
        ]
        o1 = jnp.concatenate(o_pieces, 0) * w2t
        srows = [
            jnp.sum(o1[b * D:(b + 1) * D, :], axis=0, keepdims=True)
            for b in range(B)
        ]
        out_r[q] = jnp.concatenate(srows, 0) + bo2
        return carry

    jax.lax.fori_loop(0, Q, dec_body, 0)


def _dcgru_call(Xp, todp, adjp, adjTp, W1x, W1t, b1, W2T, b2,
                WruTe, brue, WcTe, bce, WruTd, brud, WcTd, bcd,
                Wo1T, bo1, w2t, bo2, interpret=False):
    return pl.pallas_call(
        _dcgru_kernel,
        out_shape=jax.ShapeDtypeStruct((Q, B, NP), F32),
        scratch_shapes=[pltpu.VMEM((B * D, NP), F32)],
        interpret=interpret,
    )(Xp, todp, adjp, adjTp, W1x, W1t, b1, W2T, b2,
      WruTe, brue, WcTe, bce, WruTd, brud, WcTd, bcd,
      Wo1T, bo1, w2t, bo2)


def kernel(X, TE, adj_mx, W_in1, b_in1, W_in2, b_in2,
           enc_W_ru, enc_b_ru, enc_W_c, enc_b_c,
           dec_W_ru, dec_b_ru, dec_W_c, dec_b_c,
           W_out1, b_out1, W_out2, b_out2):
    f32 = F32
    ts = 2 * D

    Xsq = X[..., 0].astype(f32)
    Xp = jnp.zeros((P, B, NP), f32).at[:, :, :N].set(Xsq.transpose(1, 0, 2))
    tod = TE[:, :P, -1].astype(f32) / (12.0 * 24.0)
    todp = jnp.broadcast_to(tod.T[:, :, None], (P, B, NP))

    adjp = jnp.zeros((NP, NP), f32).at[:N, :N].set(adj_mx)
    adjTp = jnp.zeros((NP, NP), f32).at[:N, :N].set(adj_mx.T)

    W1x = W_in1[0][:, None]
    W1t = W_in1[1][:, None]
    b1 = b_in1[:, None]
    W2T = W_in2.T
    b2 = b_in2[:, None]

    WruTe = enc_W_ru.reshape(ts, M, ts).transpose(2, 1, 0).reshape(
        ts, M * ts).astype(BF16)
    brue = enc_b_ru[:, None]
    WcTe = enc_W_c.reshape(ts, M, D).transpose(2, 1, 0).reshape(
        D, M * ts).astype(BF16)
    bce = enc_b_c[:, None]
    WruTd = dec_W_ru.reshape(ts, M, ts)[D:].transpose(2, 1, 0).reshape(
        ts, M * D).astype(BF16)
    brud = dec_b_ru[:, None]
    WcTd = dec_W_c.reshape(ts, M, D)[D:].transpose(2, 1, 0).reshape(
        D, M * D).astype(BF16)
    bcd = dec_b_c[:, None]

    Wo1T = W_out1.T
    bo1 = b_out1[:, None]
    w2t = jnp.tile(W_out2[:, 0], B)[:, None]
    bo2 = b_out2.reshape(1, 1)

    out = _dcgru_call(Xp, todp, adjp, adjTp, W1x, W1t, b1, W2T, b2,
                      WruTe, brue, WcTe, bce, WruTd, brud, WcTd, bcd,
                      Wo1T, bo1, w2t, bo2)
    return out.transpose(1, 0, 2)[:, :, :N, None]
